# baseline (speedup 1.0000x reference)
import jax
import jax.numpy as jnp
from jax.experimental import pallas as pl
from jax.experimental.pallas import tpu as pltpu

_NUM_CLASS = 1000
_TILE_M = 1024
_NCH_IN = 4
_NCH_OUT = 2


def _pipeline_kernel(x_hbm, w_ref, b_ref, o_hbm,
                     x_buf, o_buf, wb_buf, in_sems, out_sems,
                     *, n_steps, tile_m):
    in_rows = tile_m // _NCH_IN
    out_rows = tile_m // _NCH_OUT

    def start_in(step, slot):
        base = step * tile_m
        for c in range(_NCH_IN):
            pltpu.make_async_copy(
                x_hbm.at[pl.ds(base + c * in_rows, in_rows), :],
                x_buf.at[slot, pl.ds(c * in_rows, in_rows), :],
                in_sems.at[slot, c],
            ).start(priority=c % 2)

    def wait_in(slot):
        for c in range(_NCH_IN):
            pltpu.make_async_copy(
                x_buf.at[slot, pl.ds(c * in_rows, in_rows), :],
                x_buf.at[slot, pl.ds(c * in_rows, in_rows), :],
                in_sems.at[slot, c],
            ).wait()

    def start_out(step, slot):
        base = step * tile_m
        for c in range(_NCH_OUT):
            pltpu.make_async_copy(
                o_buf.at[slot, pl.ds(c * out_rows, out_rows), :],
                o_hbm.at[pl.ds(base + c * out_rows, out_rows), :],
                out_sems.at[slot, c],
            ).start(priority=c % 2)

    def wait_out(slot):
        for c in range(_NCH_OUT):
            pltpu.make_async_copy(
                o_buf.at[slot, pl.ds(c * out_rows, out_rows), :],
                o_buf.at[slot, pl.ds(c * out_rows, out_rows), :],
                out_sems.at[slot, c],
            ).wait()

    wb_buf[...] = w_ref[...].astype(jnp.bfloat16)

    start_in(0, 0)
    if n_steps > 1:
        start_in(1, 1)

    for i in range(n_steps):
        slot = i % 2
        wait_in(slot)
        if i >= 2:
            wait_out(slot)
        xb = x_buf[slot].astype(jnp.bfloat16)
        acc = jnp.dot(xb, wb_buf[...], preferred_element_type=jnp.float32)
        o_buf[slot] = (acc + b_ref[...])[:, :_NUM_CLASS]
        start_out(i, slot)
        if i + 2 < n_steps:
            start_in(i + 2, slot)

    wait_out((n_steps - 2) % 2)
    wait_out((n_steps - 1) % 2)


def kernel(x, w_t, b):
    B, D = x.shape
    Dw, Cp = w_t.shape
    assert D == Dw and _NUM_CLASS <= Cp
    tile_m = min(_TILE_M, B)
    assert B % tile_m == 0
    n_steps = B // tile_m

    import functools
    body = functools.partial(_pipeline_kernel, n_steps=n_steps, tile_m=tile_m)
    return pl.pallas_call(
        body,
        out_shape=jax.ShapeDtypeStruct((B, _NUM_CLASS), x.dtype),
        in_specs=[
            pl.BlockSpec(memory_space=pltpu.MemorySpace.HBM),
            pl.BlockSpec(memory_space=pltpu.MemorySpace.VMEM),
            pl.BlockSpec(memory_space=pltpu.MemorySpace.VMEM),
        ],
        out_specs=pl.BlockSpec(memory_space=pltpu.MemorySpace.HBM),
        scratch_shapes=[
            pltpu.VMEM((2, tile_m, D), jnp.float32),
            pltpu.VMEM((2, tile_m, _NUM_CLASS), jnp.float32),
            pltpu.VMEM((D, Cp), jnp.bfloat16),
            pltpu.SemaphoreType.DMA((2, _NCH_IN)),
            pltpu.SemaphoreType.DMA((2, _NCH_OUT)),
        ],
        compiler_params=pltpu.CompilerParams(
            vmem_limit_bytes=56 * 1024 * 1024),
    )(x, w_t, b)
